# baseline (speedup 1.0000x reference)
import jax
import jax.numpy as jnp
from jax.experimental import pallas as pl
from jax.experimental.pallas import tpu as pltpu

_H_STACK = 128


def _body(sp_smem, stack_ref, hid_ref, sp_vec_ref, op_ref, out_ref, ptr_ref):
    b = pl.program_id(0)
    t = pl.program_id(1)
    tgt = sp_smem[b] + 1

    @pl.when(t == tgt)
    def _overwrite():
        out_ref[...] = hid_ref[...].reshape(out_ref.shape)

    @pl.when(t != tgt)
    def _copy():
        out_ref[...] = stack_ref[...]

    @pl.when((b == 0) & (t == 0))
    def _pointers():
        x0 = op_ref[:, 0:1]
        x1 = op_ref[:, 1:2]
        x2 = op_ref[:, 2:3]
        ops = jnp.where((x0 >= x1) & (x0 >= x2), 0,
                        jnp.where(x1 >= x2, 1, 2)).astype(jnp.int32)
        ptr_ref[...] = jnp.maximum(sp_vec_ref[...] + ops - 1, 0)


def kernel(stack, stack_pointers, stack_op, hiddens):
    B, T1, N, H = stack.shape
    sp_i32 = stack_pointers.astype(jnp.int32)

    grid_spec = pltpu.PrefetchScalarGridSpec(
        num_scalar_prefetch=1,
        grid=(B, T1),
        in_specs=[
            pl.BlockSpec((1, 1, N, H), lambda b, t, sp: (b, t, 0, 0)),
            pl.BlockSpec((1, N, _H_STACK), lambda b, t, sp: (b, 0, 0)),
            pl.BlockSpec((B, 1), lambda b, t, sp: (0, 0)),
            pl.BlockSpec((B, 3), lambda b, t, sp: (0, 0)),
        ],
        out_specs=[
            pl.BlockSpec((1, 1, N, H), lambda b, t, sp: (b, t, 0, 0)),
            pl.BlockSpec((B, 1), lambda b, t, sp: (0, 0)),
        ],
    )

    new_stack, new_ptr = pl.pallas_call(
        _body,
        grid_spec=grid_spec,
        out_shape=[
            jax.ShapeDtypeStruct((B, T1, N, H), stack.dtype),
            jax.ShapeDtypeStruct((B, 1), jnp.int32),
        ],
    )(sp_i32, stack, hiddens, sp_i32.reshape(B, 1), stack_op)

    return new_stack, new_ptr.reshape(B).astype(stack_pointers.dtype)


# speedup vs baseline: 3.3631x; 3.3631x over previous
import jax
import jax.numpy as jnp
from jax.experimental import pallas as pl
from jax.experimental.pallas import tpu as pltpu

_H_STACK = 128


def _body(sp_smem, stack_ref, hid_ref, sp_vec_ref, op_ref, out_ref, ptr_ref):
    b = pl.program_id(0)
    tgt = sp_smem[b] + 1

    out_ref[...] = stack_ref[...]
    out_ref[0, pl.ds(tgt, 1)] = hid_ref[...]

    @pl.when(b == 0)
    def _pointers():
        x0 = op_ref[:, 0:1]
        x1 = op_ref[:, 1:2]
        x2 = op_ref[:, 2:3]
        ops = jnp.where((x0 >= x1) & (x0 >= x2), 0,
                        jnp.where(x1 >= x2, 1, 2)).astype(jnp.int32)
        ptr_ref[...] = jnp.maximum(sp_vec_ref[...] + ops - 1, 0)


def kernel(stack, stack_pointers, stack_op, hiddens):
    B, T1, N, H = stack.shape
    sp_i32 = stack_pointers.astype(jnp.int32)

    grid_spec = pltpu.PrefetchScalarGridSpec(
        num_scalar_prefetch=1,
        grid=(B,),
        in_specs=[
            pl.BlockSpec((1, T1, N, H), lambda b, sp: (b, 0, 0, 0)),
            pl.BlockSpec((1, N, _H_STACK), lambda b, sp: (b, 0, 0)),
            pl.BlockSpec((B, 1), lambda b, sp: (0, 0)),
            pl.BlockSpec((B, 3), lambda b, sp: (0, 0)),
        ],
        out_specs=[
            pl.BlockSpec((1, T1, N, H), lambda b, sp: (b, 0, 0, 0)),
            pl.BlockSpec((B, 1), lambda b, sp: (0, 0)),
        ],
    )

    new_stack, new_ptr = pl.pallas_call(
        _body,
        grid_spec=grid_spec,
        out_shape=[
            jax.ShapeDtypeStruct((B, T1, N, H), stack.dtype),
            jax.ShapeDtypeStruct((B, 1), jnp.int32),
        ],
    )(sp_i32, stack, hiddens, sp_i32.reshape(B, 1), stack_op)

    return new_stack, new_ptr.reshape(B).astype(stack_pointers.dtype)
